# baseline (speedup 1.0000x reference)
import functools

import jax
import jax.numpy as jnp
from jax import lax
from jax.experimental import pallas as pl
from jax.experimental.pallas import tpu as pltpu

_S = 4


def _perm_kernel(gamma_ref, gammat_ref, noise_ref, noiset_ref, out_ref, *, n):
    gamma_row = gamma_ref[...]
    gamma_col = gammat_ref[...]
    kx = lax.broadcasted_iota(jnp.int32, (n, n), 0)
    jx = lax.broadcasted_iota(jnp.int32, (n, n), 1)
    tril = kx < jx
    kxf = kx.astype(jnp.float32)
    ones_row = jnp.ones((1, n), dtype=jnp.float32)
    for t in range(_S):
        row = gamma_row + noise_ref[t]
        col = gamma_col + noiset_ref[t]
        ltc = (col < row) | ((col == row) & tril)
        cnt = jnp.where(ltc, 1.0, 0.0)
        rank = lax.dot(ones_row, cnt)
        out_ref[t] = (kxf == rank).astype(jnp.float32)


def kernel(num_samples, gamma, gumbel_noise):
    n = gamma.shape[0]
    s = gumbel_noise.shape[0]
    gamma2d = gamma.reshape(1, n)
    gammat = gamma.reshape(n, 1)
    noise3d = gumbel_noise.reshape(s, 1, n)
    noiset3d = gumbel_noise.reshape(s, n, 1)

    return pl.pallas_call(
        functools.partial(_perm_kernel, n=n),
        grid=(s // _S,),
        in_specs=[
            pl.BlockSpec((1, n), lambda i: (0, 0)),
            pl.BlockSpec((n, 1), lambda i: (0, 0)),
            pl.BlockSpec((_S, 1, n), lambda i: (i, 0, 0)),
            pl.BlockSpec((_S, n, 1), lambda i: (i, 0, 0)),
        ],
        out_specs=pl.BlockSpec((_S, n, n), lambda i: (i, 0, 0)),
        out_shape=jax.ShapeDtypeStruct((s, n, n), jnp.float32),
        compiler_params=pltpu.CompilerParams(
            dimension_semantics=("arbitrary",),
        ),
    )(gamma2d, gammat, noise3d, noiset3d)


# speedup vs baseline: 1.1675x; 1.1675x over previous
import functools

import jax
import jax.numpy as jnp
from jax import lax
from jax.experimental import pallas as pl
from jax.experimental.pallas import tpu as pltpu

_S = 16


def _perm_kernel(gamma_ref, gammat_ref, noise_ref, noiset_ref, out_ref, *, n):
    gamma_row = gamma_ref[...]
    gamma_col = gammat_ref[...]
    kx = lax.broadcasted_iota(jnp.int32, (n, n), 0)
    jx = lax.broadcasted_iota(jnp.int32, (n, n), 1)
    tril = kx < jx
    kxf = kx.astype(jnp.float32)
    ones_row = jnp.ones((1, n), dtype=jnp.float32)
    for t in range(_S):
        row = gamma_row + noise_ref[t]
        col = gamma_col + noiset_ref[t]
        ltc = (col < row) | ((col == row) & tril)
        cnt = jnp.where(ltc, 1.0, 0.0)
        rank = lax.dot(ones_row, cnt)
        out_ref[t] = (kxf == rank).astype(jnp.float32)


def kernel(num_samples, gamma, gumbel_noise):
    n = gamma.shape[0]
    s = gumbel_noise.shape[0]
    gamma2d = gamma.reshape(1, n)
    gammat = gamma.reshape(n, 1)
    noise3d = gumbel_noise.reshape(s, 1, n)
    noiset3d = gumbel_noise.reshape(s, n, 1)

    return pl.pallas_call(
        functools.partial(_perm_kernel, n=n),
        grid=(s // _S,),
        in_specs=[
            pl.BlockSpec((1, n), lambda i: (0, 0)),
            pl.BlockSpec((n, 1), lambda i: (0, 0)),
            pl.BlockSpec((_S, 1, n), lambda i: (i, 0, 0)),
            pl.BlockSpec((_S, n, 1), lambda i: (i, 0, 0)),
        ],
        out_specs=pl.BlockSpec((_S, n, n), lambda i: (i, 0, 0)),
        out_shape=jax.ShapeDtypeStruct((s, n, n), jnp.float32),
        compiler_params=pltpu.CompilerParams(
            dimension_semantics=("arbitrary",),
        ),
    )(gamma2d, gammat, noise3d, noiset3d)
